# baseline (speedup 1.0000x reference)
import functools

import jax
import jax.numpy as jnp
from jax import lax
from jax.experimental import pallas as pl
from jax.experimental.pallas import tpu as pltpu
from jax.experimental.pallas import tpu_sc as plsc

_LANES = 16
_DEPTH = 3


@functools.lru_cache(maxsize=None)
def _build_sc_add(B, S, D, dtype):
    mesh = plsc.VectorSubcoreMesh(core_axis_name="c", subcore_axis_name="s")
    NC, NS = mesh.num_cores, mesh.num_subcores
    NW = NC * NS
    SPW = S // NW
    R = 4
    NT = SPW // R
    CH = D // _LANES
    ND = _DEPTH

    scratch = (
        [pltpu.VMEM((R, D), dtype) for _ in range(ND)]
        + [pltpu.VMEM((R, D), dtype) for _ in range(ND * B)]
        + [pltpu.VMEM((R, D), dtype) for _ in range(ND * B)]
        + [pltpu.SemaphoreType.DMA for _ in range(2 * ND)]
    )

    @functools.partial(
        pl.kernel,
        out_type=jax.ShapeDtypeStruct((B, S, D), dtype),
        mesh=mesh,
        scratch_types=scratch,
    )
    def k(x_hbm, pos_hbm, out_hbm, *scr):
        pbuf = list(scr[0:ND])
        xbuf = [list(scr[ND + sl * B: ND + (sl + 1) * B]) for sl in range(ND)]
        obuf = [list(scr[ND + ND * B + sl * B: ND + ND * B + (sl + 1) * B])
                for sl in range(ND)]
        in_sem = list(scr[ND + 2 * ND * B: 2 * ND + 2 * ND * B])
        out_sem = list(scr[2 * ND + 2 * ND * B: 3 * ND + 2 * ND * B])

        wid = lax.axis_index("s") * NC + lax.axis_index("c")
        p0 = wid * SPW

        ins, outs = {}, {}

        def issue_in(t):
            sl = t % ND
            row0 = p0 + t * R
            descs = [pltpu.async_copy(pos_hbm.at[pl.ds(row0, R)], pbuf[sl], in_sem[sl])]
            for b in range(B):
                descs.append(
                    pltpu.async_copy(x_hbm.at[b, pl.ds(row0, R)], xbuf[sl][b], in_sem[sl])
                )
            ins[t] = descs

        def issue_out(t):
            sl = t % ND
            row0 = p0 + t * R
            outs[t] = [
                pltpu.async_copy(obuf[sl][b], out_hbm.at[b, pl.ds(row0, R)], out_sem[sl])
                for b in range(B)
            ]

        def compute(t):
            sl = t % ND
            pv = pbuf[sl]
            xb = xbuf[sl]
            ob = obuf[sl]

            @plsc.parallel_loop(0, R * CH, 1, unroll=4)
            def _(i):
                r = i // CH
                col = (i % CH) * _LANES
                p = pv[r, pl.ds(col, _LANES)]
                for b in range(B):
                    ob[b][r, pl.ds(col, _LANES)] = xb[b][r, pl.ds(col, _LANES)] + p

        for t in range(ND):
            issue_in(t)
        for t in range(NT):
            for d in ins.pop(t):
                d.wait()
            if t >= ND:
                for d in outs.pop(t - ND):
                    d.wait()
            compute(t)
            issue_out(t)
            if t + ND < NT:
                issue_in(t + ND)
        for t in sorted(outs):
            for d in outs[t]:
                d.wait()

    return k


def kernel(x, pos_table):
    B, S, D = x.shape
    return _build_sc_add(B, S, D, x.dtype)(x, pos_table)


# speedup vs baseline: 1.0330x; 1.0330x over previous
import functools

import jax
import jax.numpy as jnp
from jax import lax
from jax.experimental import pallas as pl
from jax.experimental.pallas import tpu as pltpu
from jax.experimental.pallas import tpu_sc as plsc

_LANES = 16


@functools.lru_cache(maxsize=None)
def _build_sc_add(B, S, D, dtype):
    mesh = plsc.VectorSubcoreMesh(core_axis_name="c", subcore_axis_name="s")
    NC, NS = mesh.num_cores, mesh.num_subcores
    NW = NC * NS
    SPW = S // NW
    R = 8
    NT = SPW // R
    CH = D // _LANES

    scratch = (
        [pltpu.VMEM((R, D), dtype) for _ in range(2)]
        + [pltpu.VMEM((R, D), dtype) for _ in range(2 * B)]
        + [pltpu.VMEM((R, D), dtype) for _ in range(2 * B)]
        + [pltpu.SemaphoreType.DMA for _ in range(4)]
    )

    @functools.partial(
        pl.kernel,
        out_type=jax.ShapeDtypeStruct((B, S, D), dtype),
        mesh=mesh,
        scratch_types=scratch,
    )
    def k(x_hbm, pos_hbm, out_hbm, *scr):
        pbuf = [scr[0], scr[1]]
        xbuf = [[scr[2 + b] for b in range(B)], [scr[2 + B + b] for b in range(B)]]
        obuf = [[scr[2 + 2 * B + b] for b in range(B)],
                [scr[2 + 3 * B + b] for b in range(B)]]
        in_sem = [scr[2 + 4 * B], scr[3 + 4 * B]]
        out_sem = [scr[4 + 4 * B], scr[5 + 4 * B]]

        wid = lax.axis_index("s") * NC + lax.axis_index("c")
        p0 = wid * SPW

        ins, outs = {}, {}

        def issue_in(t):
            sl = t % 2
            row0 = p0 + t * R
            descs = [pltpu.async_copy(pos_hbm.at[pl.ds(row0, R)], pbuf[sl], in_sem[sl])]
            for b in range(B):
                descs.append(
                    pltpu.async_copy(x_hbm.at[b, pl.ds(row0, R)], xbuf[sl][b], in_sem[sl])
                )
            ins[t] = descs

        def issue_out(t):
            sl = t % 2
            row0 = p0 + t * R
            outs[t] = [
                pltpu.async_copy(obuf[sl][b], out_hbm.at[b, pl.ds(row0, R)], out_sem[sl])
                for b in range(B)
            ]

        def compute(t):
            sl = t % 2
            pv = pbuf[sl]
            xb = xbuf[sl]
            ob = obuf[sl]

            @plsc.parallel_loop(0, R * CH, 1, unroll=4)
            def _(i):
                r = i // CH
                col = (i % CH) * _LANES
                p = pv[r, pl.ds(col, _LANES)]
                for b in range(B):
                    ob[b][r, pl.ds(col, _LANES)] = xb[b][r, pl.ds(col, _LANES)] + p

        issue_in(0)
        issue_in(1)
        for t in range(NT):
            for d in ins.pop(t):
                d.wait()
            if t >= 2:
                for d in outs.pop(t - 2):
                    d.wait()
            compute(t)
            issue_out(t)
            if t + 2 < NT:
                issue_in(t + 2)
        for t in sorted(outs):
            for d in outs[t]:
                d.wait()

    return k


def kernel(x, pos_table):
    B, S, D = x.shape
    return _build_sc_add(B, S, D, x.dtype)(x, pos_table)


# speedup vs baseline: 1.1126x; 1.0771x over previous
import functools

import jax
import jax.numpy as jnp
from jax import lax
from jax.experimental import pallas as pl
from jax.experimental.pallas import tpu as pltpu
from jax.experimental.pallas import tpu_sc as plsc

_LANES = 16


@functools.lru_cache(maxsize=None)
def _build_sc_add(B, S, D, dtype):
    mesh = plsc.VectorSubcoreMesh(core_axis_name="c", subcore_axis_name="s")
    NC, NS = mesh.num_cores, mesh.num_subcores
    NW = NC * NS
    SPW = S // NW
    R = 8
    NT = SPW // R
    NG = NT // 2
    CH = D // _LANES

    scratch = (
        [pltpu.VMEM((R, D), dtype) for _ in range(2)]
        + [pltpu.VMEM((R, D), dtype) for _ in range(2 * B)]
        + [pltpu.VMEM((R, D), dtype) for _ in range(2 * B)]
        + [pltpu.SemaphoreType.DMA for _ in range(4)]
    )

    @functools.partial(
        pl.kernel,
        out_type=jax.ShapeDtypeStruct((B, S, D), dtype),
        mesh=mesh,
        scratch_types=scratch,
    )
    def k(x_hbm, pos_hbm, out_hbm, *scr):
        pbuf = [scr[0], scr[1]]
        xbuf = [[scr[2 + b] for b in range(B)], [scr[2 + B + b] for b in range(B)]]
        obuf = [[scr[2 + 2 * B + b] for b in range(B)],
                [scr[2 + 3 * B + b] for b in range(B)]]
        in_sem = [scr[2 + 4 * B], scr[3 + 4 * B]]
        out_sem = [scr[4 + 4 * B], scr[5 + 4 * B]]

        wid = lax.axis_index("s") * NC + lax.axis_index("c")
        p0 = wid * SPW

        def issue_in(sl, row0):
            pltpu.async_copy(pos_hbm.at[pl.ds(row0, R)], pbuf[sl], in_sem[sl])
            for b in range(B):
                pltpu.async_copy(x_hbm.at[b, pl.ds(row0, R)], xbuf[sl][b], in_sem[sl])

        def issue_out(sl, row0):
            for b in range(B):
                pltpu.async_copy(obuf[sl][b], out_hbm.at[b, pl.ds(row0, R)], out_sem[sl])

        def wait_in(sl):
            pltpu.make_async_copy(pos_hbm.at[pl.ds(0, R)], pbuf[sl], in_sem[sl]).wait()
            for b in range(B):
                pltpu.make_async_copy(
                    x_hbm.at[b, pl.ds(0, R)], xbuf[sl][b], in_sem[sl]
                ).wait()

        def wait_out(sl):
            for b in range(B):
                pltpu.make_async_copy(
                    x_hbm.at[b, pl.ds(0, R)], obuf[sl][b], out_sem[sl]
                ).wait()

        def compute(sl):
            pv = pbuf[sl]
            xb = xbuf[sl]
            ob = obuf[sl]

            @plsc.parallel_loop(0, R * CH, 1, unroll=4)
            def _(i):
                r = i // CH
                col = (i % CH) * _LANES
                p = pv[r, pl.ds(col, _LANES)]
                for b in range(B):
                    ob[b][r, pl.ds(col, _LANES)] = xb[b][r, pl.ds(col, _LANES)] + p

        issue_in(0, p0)
        issue_in(1, p0 + R)
        wait_in(0)
        compute(0)
        issue_out(0, p0)
        issue_in(0, p0 + 2 * R)
        wait_in(1)
        compute(1)
        issue_out(1, p0 + R)
        issue_in(1, p0 + 3 * R)

        def pair(g, carry):
            row0 = p0 + (2 * g) * R
            for sl in range(2):
                rw = row0 + sl * R
                wait_in(sl)
                wait_out(sl)
                compute(sl)
                issue_out(sl, rw)
                issue_in(sl, rw + 2 * R)
            return carry

        lax.fori_loop(1, NG - 1, pair, 0)

        rlast = p0 + (NT - 2) * R
        for sl in range(2):
            wait_in(sl)
            wait_out(sl)
            compute(sl)
            issue_out(sl, rlast + sl * R)
        wait_out(0)
        wait_out(1)

    return k


def kernel(x, pos_table):
    B, S, D = x.shape
    return _build_sc_add(B, S, D, x.dtype)(x, pos_table)
